# baseline (speedup 1.0000x reference)
import functools

import jax
import jax.numpy as jnp
from jax import lax
from jax.experimental import pallas as pl
from jax.experimental.pallas import tpu as pltpu
from jax.experimental.pallas import tpu_sc as plsc

N = 10000
NPAD = 10240
M = 320000
D_IN = 128
D_OUT = 129
DP = 144
NS = 16
EDGES_PER_TILE = M // NS
CHUNK = 40
N_CHUNKS = EDGES_PER_TILE // CHUNK
NBUF = 5
N_GROUPS = N_CHUNKS // NBUF
ROWS_PER_TILE = NPAD // NS

_MESH = plsc.VectorSubcoreMesh(
    core_axis_name="c", subcore_axis_name="s", num_cores=1)


@functools.partial(
    pl.kernel,
    out_type=jax.ShapeDtypeStruct((NPAD, DP), jnp.float32),
    mesh=_MESH,
    scratch_types=[
        pltpu.VMEM((NBUF, CHUNK), jnp.int32),
        pltpu.VMEM((NBUF, CHUNK), jnp.int32),
        pltpu.VMEM((NBUF, CHUNK, DP), jnp.float32),
        pltpu.VMEM_SHARED((NPAD, DP), jnp.float32),
        pltpu.SemaphoreType.DMA((NBUF,)),
        pltpu.SemaphoreType.DMA((NBUF,)),
        pltpu.SemaphoreType.DMA((NBUF,)),
    ],
    compiler_params=pltpu.CompilerParams(use_tc_tiling_on_sc=False),
)
def _sc_gather_scatter_add(node_hbm, src_hbm, dst_hbm, out_hbm,
                           src_v, dst_v, rows_v, acc, sem_i, sem_g, sem_a):
    s = lax.axis_index("s")
    row0 = pl.multiple_of(s * ROWS_PER_TILE, 8)

    pltpu.sync_copy(node_hbm.at[pl.ds(row0, ROWS_PER_TILE)],
                    acc.at[pl.ds(row0, ROWS_PER_TILE)])
    plsc.subcore_barrier()

    e0 = s * EDGES_PER_TILE

    def _idx_copies(chunk, b):
        base = e0 + chunk * CHUNK
        return (
            pltpu.make_async_copy(src_hbm.at[pl.ds(base, CHUNK)],
                                  src_v.at[b], sem_i.at[b]),
            pltpu.make_async_copy(dst_hbm.at[pl.ds(base, CHUNK)],
                                  dst_v.at[b], sem_i.at[b]),
        )

    for b in range(NBUF):
        for d in _idx_copies(b, b):
            d.start()

    def group(g, carry):
        gathers = []
        for b in range(NBUF):
            for d in _idx_copies(g * NBUF + b, b):
                d.wait()
            d = pltpu.make_async_copy(node_hbm.at[src_v.at[b]],
                                      rows_v.at[b], sem_g.at[b])
            d.start()
            gathers.append(d)
        scatters = []
        for b in range(NBUF):
            gathers[b].wait()
            d = pltpu.async_copy(rows_v.at[b], acc.at[dst_v.at[b]],
                                 sem_a.at[b], add=True)
            scatters.append(d)
        for b in range(NBUF):
            scatters[b].wait()

            @pl.when(g < N_GROUPS - 1)
            def _():
                for d in _idx_copies((g + 1) * NBUF + b, b):
                    d.start()

        return carry

    lax.fori_loop(0, N_GROUPS, group, 0)

    plsc.subcore_barrier()

    pltpu.sync_copy(acc.at[pl.ds(row0, ROWS_PER_TILE)],
                    out_hbm.at[pl.ds(row0, ROWS_PER_TILE)])


def _slice_body(a_ref, o_ref):
    o_ref[...] = a_ref[:, :D_OUT]


_slice_cols = pl.pallas_call(
    _slice_body,
    grid=(5,),
    in_specs=[pl.BlockSpec((N // 5, DP), lambda i: (i, 0))],
    out_specs=pl.BlockSpec((N // 5, D_OUT), lambda i: (i, 0)),
    out_shape=jax.ShapeDtypeStruct((N, D_OUT), jnp.float32),
)


def kernel(node, edge_index, eps_k):
    del eps_k
    rkey = jax.random.fold_in(jax.random.key(0), 42)
    rand = jax.random.uniform(
        rkey, (N, 1), minval=0.0, maxval=100.0, dtype=jnp.float32) / 100.0
    node_pad = jnp.zeros((NPAD, DP), jnp.float32)
    node_pad = node_pad.at[:N, :D_IN].set(node)
    node_pad = node_pad.at[:N, D_IN:D_OUT].set(rand)
    dst = edge_index[0]
    src = edge_index[1]
    acc = _sc_gather_scatter_add(node_pad, src, dst)
    return _slice_cols(acc)
